# baseline (speedup 1.0000x reference)
import jax
import jax.numpy as jnp
from jax.experimental import pallas as pl
from jax.experimental.pallas import tpu as pltpu

_CP = pltpu.CompilerParams(vmem_limit_bytes=64 * 1024 * 1024)

_ALPHA = 0.1
_BF = jnp.bfloat16
_F8 = jnp.float8_e4m3fn


def _pick_blk(n, want):
    return want if n % want == 0 else n


def _dot(a, b):
    return jnp.dot(a, b, preferred_element_type=jnp.float32)


def _lrelu(x):
    return jnp.where(x >= 0.0, x, _ALPHA * x)


def _mm_body(x_ref, w_ref, o_ref):
    o_ref[...] = _dot(x_ref[...], w_ref[...]).astype(o_ref.dtype)


def _mm(x, w):
    n, d = x.shape
    h = w.shape[1]
    blk = _pick_blk(n, 1000)
    return pl.pallas_call(
        _mm_body,
        grid=(n // blk,),
        in_specs=[
            pl.BlockSpec((blk, d), lambda i: (i, 0)),
            pl.BlockSpec((d, h), lambda i: (0, 0)),
        ],
        out_specs=pl.BlockSpec((blk, h), lambda i: (i, 0)),
        compiler_params=_CP,
        out_shape=jax.ShapeDtypeStruct((n, h), _BF),
    )(x, w)


def _spmm_a_body(adj_ref, y_ref, b_ref, wn_ref, yn_ref, adjb_ref):
    a = adj_ref[...].astype(_BF)
    h = _lrelu(_dot(a, y_ref[...]) + b_ref[...])
    yn_ref[...] = _dot(h, wn_ref[...]).astype(_BF)
    adjb_ref[...] = a.astype(_F8)


def _spmm_a(adj, y, b, w_next):
    m, k = adj.shape
    h = y.shape[1]
    hn = w_next.shape[1]
    blk = _pick_blk(m, 400)
    return pl.pallas_call(
        _spmm_a_body,
        grid=(m // blk,),
        in_specs=[
            pl.BlockSpec((blk, k), lambda i: (i, 0)),
            pl.BlockSpec((k, h), lambda i: (0, 0)),
            pl.BlockSpec((1, h), lambda i: (0, 0)),
            pl.BlockSpec((h, hn), lambda i: (0, 0)),
        ],
        out_specs=[
            pl.BlockSpec((blk, hn), lambda i: (i, 0)),
            pl.BlockSpec((blk, k), lambda i: (i, 0)),
        ],
        compiler_params=_CP,
        out_shape=[
            jax.ShapeDtypeStruct((m, hn), _BF),
            jax.ShapeDtypeStruct((m, k), _F8),
        ],
    )(adj, y, b.reshape(1, h), w_next)


def _spmm_b_body(adj_ref, yh_ref, yl_ref, ih_ref, il_ref, b_ref, feat_ref,
                 wu1_ref, wu2_ref, bu_ref, wn_ref, u_ref, yn_ref):
    a = adj_ref[...]
    acc = _dot(a, yh_ref[...]) * ih_ref[...] + _dot(a, yl_ref[...]) * il_ref[...]
    h = _lrelu(acc + b_ref[...])
    u = _dot(h, wu1_ref[...]) + _dot(feat_ref[...], wu2_ref[...]) + bu_ref[...]
    u = jnp.maximum(u, 0.0)
    u_ref[...] = u
    yn_ref[...] = _dot(u, wn_ref[...]).astype(_BF)


def _spmm_b(adj_bf, yh, yl, ih, il, b, feat, wu, bu, w_next):
    m, k = adj_bf.shape
    h = yh.shape[1]
    df = feat.shape[1]
    hu = wu.shape[1]
    hn = w_next.shape[1]
    blk = _pick_blk(m, 1000)
    return pl.pallas_call(
        _spmm_b_body,
        grid=(m // blk,),
        in_specs=[
            pl.BlockSpec((blk, k), lambda i: (i, 0)),
            pl.BlockSpec((k, h), lambda i: (0, 0)),
            pl.BlockSpec((k, h), lambda i: (0, 0)),
            pl.BlockSpec((1, 1), lambda i: (0, 0)),
            pl.BlockSpec((1, 1), lambda i: (0, 0)),
            pl.BlockSpec((1, h), lambda i: (0, 0)),
            pl.BlockSpec((blk, df), lambda i: (i, 0)),
            pl.BlockSpec((h, hu), lambda i: (0, 0)),
            pl.BlockSpec((df, hu), lambda i: (0, 0)),
            pl.BlockSpec((1, hu), lambda i: (0, 0)),
            pl.BlockSpec((hu, hn), lambda i: (0, 0)),
        ],
        out_specs=[
            pl.BlockSpec((blk, hu), lambda i: (i, 0)),
            pl.BlockSpec((blk, hn), lambda i: (i, 0)),
        ],
        compiler_params=_CP,
        out_shape=[
            jax.ShapeDtypeStruct((m, hu), jnp.float32),
            jax.ShapeDtypeStruct((m, hn), _BF),
        ],
    )(adj_bf, yh, yl, ih, il, b.reshape(1, h), feat, wu[:h], wu[h:],
      bu.reshape(1, hu), w_next)


def _spmm_c_body(adj_ref, yh_ref, yl_ref, ih_ref, il_ref, b_ref, wn_ref,
                 yn_ref):
    a = adj_ref[...]
    acc = _dot(a, yh_ref[...]) * ih_ref[...] + _dot(a, yl_ref[...]) * il_ref[...]
    h = _lrelu(acc + b_ref[...])
    yn_ref[...] = _dot(h, wn_ref[...]).astype(_BF)


def _spmm_c(adj_bf, yh, yl, ih, il, b, w_next):
    m, k = adj_bf.shape
    y = yh
    h = y.shape[1]
    hn = w_next.shape[1]
    blk = _pick_blk(m, 1000)
    return pl.pallas_call(
        _spmm_c_body,
        grid=(m // blk,),
        in_specs=[
            pl.BlockSpec((blk, k), lambda i: (i, 0)),
            pl.BlockSpec((k, h), lambda i: (0, 0)),
            pl.BlockSpec((k, h), lambda i: (0, 0)),
            pl.BlockSpec((1, 1), lambda i: (0, 0)),
            pl.BlockSpec((1, 1), lambda i: (0, 0)),
            pl.BlockSpec((1, h), lambda i: (0, 0)),
            pl.BlockSpec((h, hn), lambda i: (0, 0)),
        ],
        out_specs=pl.BlockSpec((blk, hn), lambda i: (i, 0)),
        compiler_params=_CP,
        out_shape=jax.ShapeDtypeStruct((m, hn), _BF),
    )(adj_bf, yh, yl, ih, il, b.reshape(1, h), w_next)


def _spmm_d_body(adj_ref, yh_ref, yl_ref, ih_ref, il_ref, b_ref, feat_ref,
                 wu1_ref, wu2_ref, bu_ref, o_ref):
    a = adj_ref[...]
    acc = _dot(a, yh_ref[...]) * ih_ref[...] + _dot(a, yl_ref[...]) * il_ref[...]
    h = _lrelu(acc + b_ref[...])
    o_ref[...] = (_dot(h, wu1_ref[...]) + _dot(feat_ref[...], wu2_ref[...])
                  + bu_ref[...])


def _spmm_d(adj_bf, yh, yl, ih, il, b, feat, wu, bu):
    m, k = adj_bf.shape
    y = yh
    h = y.shape[1]
    df = feat.shape[1]
    hu = wu.shape[1]
    blk = _pick_blk(m, 1000)
    return pl.pallas_call(
        _spmm_d_body,
        grid=(m // blk,),
        in_specs=[
            pl.BlockSpec((blk, k), lambda i: (i, 0)),
            pl.BlockSpec((k, h), lambda i: (0, 0)),
            pl.BlockSpec((k, h), lambda i: (0, 0)),
            pl.BlockSpec((1, 1), lambda i: (0, 0)),
            pl.BlockSpec((1, 1), lambda i: (0, 0)),
            pl.BlockSpec((1, h), lambda i: (0, 0)),
            pl.BlockSpec((blk, df), lambda i: (i, 0)),
            pl.BlockSpec((h, hu), lambda i: (0, 0)),
            pl.BlockSpec((df, hu), lambda i: (0, 0)),
            pl.BlockSpec((1, hu), lambda i: (0, 0)),
        ],
        out_specs=pl.BlockSpec((blk, hu), lambda i: (i, 0)),
        compiler_params=_CP,
        out_shape=jax.ShapeDtypeStruct((m, hu), jnp.float32),
    )(adj_bf, yh, yl, ih, il, b.reshape(1, h), feat, wu[:h], wu[h:],
      bu.reshape(1, hu))


def _q8(y):
    y32 = y.astype(jnp.float32)
    amax = jnp.max(jnp.abs(y32))
    s = 448.0 / jnp.maximum(amax, 1e-30)
    ys = y32 * s
    qh = ys.astype(_F8)
    ql = ((ys - qh.astype(jnp.float32)) * 28.0).astype(_F8)
    inv = (1.0 / s).reshape(1, 1)
    return qh, ql, inv, inv / 28.0


def kernel(ufea, vfea, UV_adj, VU_adj, d_gc1_w, d_gc1_b, d_gc2_w, d_gc2_b, d_gc3_w, d_gc3_b, d_gc4_w, d_gc4_b, l_gc1_w, l_gc1_b, l_gc2_w, l_gc2_b, l_gc3m_w, l_gc3m_b, l_gc3s_w, l_gc3s_b, l_gc4m_w, l_gc4m_b, l_gc4s_w, l_gc4s_b, d_uu_w, d_uu_b, d_iu_w, d_iu_b, l_uum_w, l_uum_b, l_uus_w, l_uus_b, l_ium_w, l_ium_b, l_ius_w, l_ius_b):
    y1 = _mm(ufea, d_gc1_w)
    y2 = _mm(vfea, d_gc2_w)
    y3, VU_bf = _spmm_a(VU_adj, y1, d_gc1_b, d_gc3_w)
    y4, UV_bf = _spmm_a(UV_adj, y2, d_gc2_b, d_gc4_w)
    u, y5 = _spmm_b(UV_bf, *_q8(y3), d_gc3_b, ufea, d_uu_w, d_uu_b, l_gc1_w)
    v, y6 = _spmm_b(VU_bf, *_q8(y4), d_gc4_b, vfea, d_iu_w, d_iu_b, l_gc2_w)
    y7 = _spmm_c(VU_bf, *_q8(y5), l_gc1_b, l_gc3m_w)
    y8 = _spmm_c(UV_bf, *_q8(y6), l_gc2_b, l_gc4m_w)
    user = _spmm_d(UV_bf, *_q8(y7), l_gc3m_b, u, l_uum_w, l_uum_b)
    item = _spmm_d(VU_bf, *_q8(y8), l_gc4m_b, v, l_ium_w, l_ium_b)
    return user, item


# speedup vs baseline: 1.0581x; 1.0581x over previous
import jax
import jax.numpy as jnp
from jax.experimental import pallas as pl
from jax.experimental.pallas import tpu as pltpu

_CP = pltpu.CompilerParams(vmem_limit_bytes=64 * 1024 * 1024)

_ALPHA = 0.1
_BF = jnp.bfloat16
_F8 = jnp.float8_e4m3fn


def _pick_blk(n, want):
    return want if n % want == 0 else n


def _dot(a, b):
    return jnp.dot(a, b, preferred_element_type=jnp.float32)


def _lrelu(x):
    return jnp.where(x >= 0.0, x, _ALPHA * x)


def _mm_body(x_ref, w_ref, o_ref):
    o_ref[...] = _dot(x_ref[...], w_ref[...]).astype(o_ref.dtype)


def _mm(x, w):
    n, d = x.shape
    h = w.shape[1]
    blk = _pick_blk(n, 1000)
    return pl.pallas_call(
        _mm_body,
        grid=(n // blk,),
        in_specs=[
            pl.BlockSpec((blk, d), lambda i: (i, 0)),
            pl.BlockSpec((d, h), lambda i: (0, 0)),
        ],
        out_specs=pl.BlockSpec((blk, h), lambda i: (i, 0)),
        compiler_params=_CP,
        out_shape=jax.ShapeDtypeStruct((n, h), _BF),
    )(x, w)


def _spmm_a_body(adj_ref, y_ref, b_ref, wn_ref, yn_ref, adjb_ref):
    a = adj_ref[...].astype(_BF)
    h = _lrelu(_dot(a, y_ref[...]) + b_ref[...])
    yn_ref[...] = _dot(h, wn_ref[...]).astype(_BF)
    adjb_ref[...] = a.astype(_F8)


def _spmm_a(adj, y, b, w_next):
    m, k = adj.shape
    h = y.shape[1]
    hn = w_next.shape[1]
    blk = _pick_blk(m, 400)
    return pl.pallas_call(
        _spmm_a_body,
        grid=(m // blk,),
        in_specs=[
            pl.BlockSpec((blk, k), lambda i: (i, 0)),
            pl.BlockSpec((k, h), lambda i: (0, 0)),
            pl.BlockSpec((1, h), lambda i: (0, 0)),
            pl.BlockSpec((h, hn), lambda i: (0, 0)),
        ],
        out_specs=[
            pl.BlockSpec((blk, hn), lambda i: (i, 0)),
            pl.BlockSpec((blk, k), lambda i: (i, 0)),
        ],
        compiler_params=_CP,
        out_shape=[
            jax.ShapeDtypeStruct((m, hn), _BF),
            jax.ShapeDtypeStruct((m, k), _F8),
        ],
    )(adj, y, b.reshape(1, h), w_next)


def _spmm_b_body(adj_ref, y_ref, b_ref, feat_ref, wu1_ref, wu2_ref, bu_ref,
                 wn_ref, u_ref, yn_ref):
    h = _lrelu(_dot(adj_ref[...], y_ref[...]) + b_ref[...])
    u = _dot(h, wu1_ref[...]) + _dot(feat_ref[...], wu2_ref[...]) + bu_ref[...]
    u = jnp.maximum(u, 0.0)
    u_ref[...] = u
    yn_ref[...] = _dot(u, wn_ref[...]).astype(_BF)


def _spmm_b(adj_bf, y, b, feat, wu, bu, w_next):
    m, k = adj_bf.shape
    h = y.shape[1]
    df = feat.shape[1]
    hu = wu.shape[1]
    hn = w_next.shape[1]
    blk = _pick_blk(m, 1000)
    return pl.pallas_call(
        _spmm_b_body,
        grid=(m // blk,),
        in_specs=[
            pl.BlockSpec((blk, k), lambda i: (i, 0)),
            pl.BlockSpec((k, h), lambda i: (0, 0)),
            pl.BlockSpec((1, h), lambda i: (0, 0)),
            pl.BlockSpec((blk, df), lambda i: (i, 0)),
            pl.BlockSpec((h, hu), lambda i: (0, 0)),
            pl.BlockSpec((df, hu), lambda i: (0, 0)),
            pl.BlockSpec((1, hu), lambda i: (0, 0)),
            pl.BlockSpec((hu, hn), lambda i: (0, 0)),
        ],
        out_specs=[
            pl.BlockSpec((blk, hu), lambda i: (i, 0)),
            pl.BlockSpec((blk, hn), lambda i: (i, 0)),
        ],
        compiler_params=_CP,
        out_shape=[
            jax.ShapeDtypeStruct((m, hu), jnp.float32),
            jax.ShapeDtypeStruct((m, hn), _BF),
        ],
    )(adj_bf, y, b.reshape(1, h), feat, wu[:h], wu[h:], bu.reshape(1, hu),
      w_next)


def _spmm_c_body(adj_ref, y_ref, b_ref, wn_ref, yn_ref):
    h = _lrelu(_dot(adj_ref[...], y_ref[...]) + b_ref[...])
    yn_ref[...] = _dot(h, wn_ref[...]).astype(_BF)


def _spmm_c(adj_bf, y, b, w_next):
    m, k = adj_bf.shape
    h = y.shape[1]
    hn = w_next.shape[1]
    blk = _pick_blk(m, 1000)
    return pl.pallas_call(
        _spmm_c_body,
        grid=(m // blk,),
        in_specs=[
            pl.BlockSpec((blk, k), lambda i: (i, 0)),
            pl.BlockSpec((k, h), lambda i: (0, 0)),
            pl.BlockSpec((1, h), lambda i: (0, 0)),
            pl.BlockSpec((h, hn), lambda i: (0, 0)),
        ],
        out_specs=pl.BlockSpec((blk, hn), lambda i: (i, 0)),
        compiler_params=_CP,
        out_shape=jax.ShapeDtypeStruct((m, hn), _BF),
    )(adj_bf, y, b.reshape(1, h), w_next)


def _spmm_d_body(adj_ref, y_ref, b_ref, feat_ref, wu1_ref, wu2_ref, bu_ref,
                 o_ref):
    h = _lrelu(_dot(adj_ref[...], y_ref[...]) + b_ref[...])
    o_ref[...] = (_dot(h, wu1_ref[...]) + _dot(feat_ref[...], wu2_ref[...])
                  + bu_ref[...])


def _spmm_d(adj_bf, y, b, feat, wu, bu):
    m, k = adj_bf.shape
    h = y.shape[1]
    df = feat.shape[1]
    hu = wu.shape[1]
    blk = _pick_blk(m, 1000)
    return pl.pallas_call(
        _spmm_d_body,
        grid=(m // blk,),
        in_specs=[
            pl.BlockSpec((blk, k), lambda i: (i, 0)),
            pl.BlockSpec((k, h), lambda i: (0, 0)),
            pl.BlockSpec((1, h), lambda i: (0, 0)),
            pl.BlockSpec((blk, df), lambda i: (i, 0)),
            pl.BlockSpec((h, hu), lambda i: (0, 0)),
            pl.BlockSpec((df, hu), lambda i: (0, 0)),
            pl.BlockSpec((1, hu), lambda i: (0, 0)),
        ],
        out_specs=pl.BlockSpec((blk, hu), lambda i: (i, 0)),
        compiler_params=_CP,
        out_shape=jax.ShapeDtypeStruct((m, hu), jnp.float32),
    )(adj_bf, y, b.reshape(1, h), feat, wu[:h], wu[h:], bu.reshape(1, hu))


def _spmm_a2_body(adjP_ref, adjQ_ref, yP_ref, yQ_ref, bP_ref, bQ_ref,
                  wnP_ref, wnQ_ref, ynP_ref, cP_ref, ynQ_ref, cQ_ref):
    s = pl.num_programs(0) // 2
    i = pl.program_id(0)

    @pl.when(i < s)
    def _p():
        a = adjP_ref[...].astype(_BF)
        h = _lrelu(_dot(a, yP_ref[...]) + bP_ref[...])
        ynP_ref[...] = _dot(h, wnP_ref[...]).astype(_BF)
        cP_ref[...] = a.astype(_F8)

    @pl.when(i >= s)
    def _q():
        a = adjQ_ref[...].astype(_BF)
        h = _lrelu(_dot(a, yQ_ref[...]) + bQ_ref[...])
        ynQ_ref[...] = _dot(h, wnQ_ref[...]).astype(_BF)
        cQ_ref[...] = a.astype(_F8)


def _spmm_a2(adjP, adjQ, yP, yQ, bP, bQ, wnP, wnQ):
    m, k = adjP.shape
    h = yP.shape[1]
    hn = wnP.shape[1]
    blk = _pick_blk(m, 200)
    s = m // blk
    const = lambda i: (0, 0)
    padv = lambda i: (jnp.minimum(i, s - 1), 0)
    qadv = lambda i: (jnp.maximum(i - s, 0), 0)
    return pl.pallas_call(
        _spmm_a2_body,
        grid=(2 * s,),
        in_specs=[
            pl.BlockSpec((blk, k), padv),
            pl.BlockSpec((blk, k), qadv),
            pl.BlockSpec((k, h), const),
            pl.BlockSpec((k, h), const),
            pl.BlockSpec((1, h), const),
            pl.BlockSpec((1, h), const),
            pl.BlockSpec((h, hn), const),
            pl.BlockSpec((h, hn), const),
        ],
        out_specs=[
            pl.BlockSpec((blk, hn), padv),
            pl.BlockSpec((blk, k), padv),
            pl.BlockSpec((blk, hn), qadv),
            pl.BlockSpec((blk, k), qadv),
        ],
        compiler_params=_CP,
        out_shape=[
            jax.ShapeDtypeStruct((m, hn), _BF),
            jax.ShapeDtypeStruct((m, k), _F8),
            jax.ShapeDtypeStruct((m, hn), _BF),
            jax.ShapeDtypeStruct((m, k), _F8),
        ],
    )(adjP, adjQ, yP, yQ, bP.reshape(1, h), bQ.reshape(1, h), wnP, wnQ)


def _spmm_b2_body(adjP_ref, adjQ_ref, yP_ref, yQ_ref, bP_ref, bQ_ref,
                  fP_ref, fQ_ref, wu1P_ref, wu2P_ref, buP_ref,
                  wu1Q_ref, wu2Q_ref, buQ_ref, wnP_ref, wnQ_ref,
                  uP_ref, ynP_ref, uQ_ref, ynQ_ref):
    s = pl.num_programs(0) // 2
    i = pl.program_id(0)

    @pl.when(i < s)
    def _p():
        h = _lrelu(_dot(adjP_ref[...], yP_ref[...]) + bP_ref[...])
        u = (_dot(h, wu1P_ref[...]) + _dot(fP_ref[...], wu2P_ref[...])
             + buP_ref[...])
        u = jnp.maximum(u, 0.0)
        uP_ref[...] = u
        ynP_ref[...] = _dot(u, wnP_ref[...]).astype(_BF)

    @pl.when(i >= s)
    def _q():
        h = _lrelu(_dot(adjQ_ref[...], yQ_ref[...]) + bQ_ref[...])
        u = (_dot(h, wu1Q_ref[...]) + _dot(fQ_ref[...], wu2Q_ref[...])
             + buQ_ref[...])
        u = jnp.maximum(u, 0.0)
        uQ_ref[...] = u
        ynQ_ref[...] = _dot(u, wnQ_ref[...]).astype(_BF)


def _spmm_b2(adjP, adjQ, yP, yQ, bP, bQ, fP, fQ, wuP, buP, wuQ, buQ,
             wnP, wnQ):
    m, k = adjP.shape
    h = yP.shape[1]
    df = fP.shape[1]
    hu = wuP.shape[1]
    hn = wnP.shape[1]
    blk = _pick_blk(m, 1000)
    s = m // blk
    const = lambda i: (0, 0)
    padv = lambda i: (jnp.minimum(i, s - 1), 0)
    qadv = lambda i: (jnp.maximum(i - s, 0), 0)
    return pl.pallas_call(
        _spmm_b2_body,
        grid=(2 * s,),
        in_specs=[
            pl.BlockSpec((blk, k), padv),
            pl.BlockSpec((blk, k), qadv),
            pl.BlockSpec((k, h), const),
            pl.BlockSpec((k, h), const),
            pl.BlockSpec((1, h), const),
            pl.BlockSpec((1, h), const),
            pl.BlockSpec((blk, df), padv),
            pl.BlockSpec((blk, df), qadv),
            pl.BlockSpec((h, hu), const),
            pl.BlockSpec((df, hu), const),
            pl.BlockSpec((1, hu), const),
            pl.BlockSpec((h, hu), const),
            pl.BlockSpec((df, hu), const),
            pl.BlockSpec((1, hu), const),
            pl.BlockSpec((hu, hn), const),
            pl.BlockSpec((hu, hn), const),
        ],
        out_specs=[
            pl.BlockSpec((blk, hu), padv),
            pl.BlockSpec((blk, hn), padv),
            pl.BlockSpec((blk, hu), qadv),
            pl.BlockSpec((blk, hn), qadv),
        ],
        compiler_params=_CP,
        out_shape=[
            jax.ShapeDtypeStruct((m, hu), jnp.float32),
            jax.ShapeDtypeStruct((m, hn), _BF),
            jax.ShapeDtypeStruct((m, hu), jnp.float32),
            jax.ShapeDtypeStruct((m, hn), _BF),
        ],
    )(adjP, adjQ, yP, yQ, bP.reshape(1, h), bQ.reshape(1, h), fP, fQ,
      wuP[:h], wuP[h:], buP.reshape(1, hu), wuQ[:h], wuQ[h:],
      buQ.reshape(1, hu), wnP, wnQ)


def _spmm_c2_body(adjP_ref, adjQ_ref, yP_ref, yQ_ref, bP_ref, bQ_ref,
                  wnP_ref, wnQ_ref, ynP_ref, ynQ_ref):
    s = pl.num_programs(0) // 2
    i = pl.program_id(0)

    @pl.when(i < s)
    def _p():
        h = _lrelu(_dot(adjP_ref[...], yP_ref[...]) + bP_ref[...])
        ynP_ref[...] = _dot(h, wnP_ref[...]).astype(_BF)

    @pl.when(i >= s)
    def _q():
        h = _lrelu(_dot(adjQ_ref[...], yQ_ref[...]) + bQ_ref[...])
        ynQ_ref[...] = _dot(h, wnQ_ref[...]).astype(_BF)


def _spmm_c2(adjP, adjQ, yP, yQ, bP, bQ, wnP, wnQ):
    m, k = adjP.shape
    h = yP.shape[1]
    hn = wnP.shape[1]
    blk = _pick_blk(m, 1000)
    s = m // blk
    const = lambda i: (0, 0)
    padv = lambda i: (jnp.minimum(i, s - 1), 0)
    qadv = lambda i: (jnp.maximum(i - s, 0), 0)
    return pl.pallas_call(
        _spmm_c2_body,
        grid=(2 * s,),
        in_specs=[
            pl.BlockSpec((blk, k), padv),
            pl.BlockSpec((blk, k), qadv),
            pl.BlockSpec((k, h), const),
            pl.BlockSpec((k, h), const),
            pl.BlockSpec((1, h), const),
            pl.BlockSpec((1, h), const),
            pl.BlockSpec((h, hn), const),
            pl.BlockSpec((h, hn), const),
        ],
        out_specs=[
            pl.BlockSpec((blk, hn), padv),
            pl.BlockSpec((blk, hn), qadv),
        ],
        compiler_params=_CP,
        out_shape=[
            jax.ShapeDtypeStruct((m, hn), _BF),
            jax.ShapeDtypeStruct((m, hn), _BF),
        ],
    )(adjP, adjQ, yP, yQ, bP.reshape(1, h), bQ.reshape(1, h), wnP, wnQ)


def _spmm_d2_body(adjP_ref, adjQ_ref, yP_ref, yQ_ref, bP_ref, bQ_ref,
                  fP_ref, fQ_ref, wu1P_ref, wu2P_ref, buP_ref,
                  wu1Q_ref, wu2Q_ref, buQ_ref, oP_ref, oQ_ref):
    s = pl.num_programs(0) // 2
    i = pl.program_id(0)

    @pl.when(i < s)
    def _p():
        h = _lrelu(_dot(adjP_ref[...], yP_ref[...]) + bP_ref[...])
        oP_ref[...] = (_dot(h, wu1P_ref[...])
                       + _dot(fP_ref[...], wu2P_ref[...]) + buP_ref[...])

    @pl.when(i >= s)
    def _q():
        h = _lrelu(_dot(adjQ_ref[...], yQ_ref[...]) + bQ_ref[...])
        oQ_ref[...] = (_dot(h, wu1Q_ref[...])
                       + _dot(fQ_ref[...], wu2Q_ref[...]) + buQ_ref[...])


def _spmm_d2(adjP, adjQ, yP, yQ, bP, bQ, fP, fQ, wuP, wuQ, buP, buQ):
    m, k = adjP.shape
    h = yP.shape[1]
    df = fP.shape[1]
    hu = wuP.shape[1]
    blk = _pick_blk(m, 1000)
    s = m // blk
    const = lambda i: (0, 0)
    padv = lambda i: (jnp.minimum(i, s - 1), 0)
    qadv = lambda i: (jnp.maximum(i - s, 0), 0)
    return pl.pallas_call(
        _spmm_d2_body,
        grid=(2 * s,),
        in_specs=[
            pl.BlockSpec((blk, k), padv),
            pl.BlockSpec((blk, k), qadv),
            pl.BlockSpec((k, h), const),
            pl.BlockSpec((k, h), const),
            pl.BlockSpec((1, h), const),
            pl.BlockSpec((1, h), const),
            pl.BlockSpec((blk, df), padv),
            pl.BlockSpec((blk, df), qadv),
            pl.BlockSpec((h, hu), const),
            pl.BlockSpec((df, hu), const),
            pl.BlockSpec((1, hu), const),
            pl.BlockSpec((h, hu), const),
            pl.BlockSpec((df, hu), const),
            pl.BlockSpec((1, hu), const),
        ],
        out_specs=[
            pl.BlockSpec((blk, hu), padv),
            pl.BlockSpec((blk, hu), qadv),
        ],
        compiler_params=_CP,
        out_shape=[
            jax.ShapeDtypeStruct((m, hu), jnp.float32),
            jax.ShapeDtypeStruct((m, hu), jnp.float32),
        ],
    )(adjP, adjQ, yP, yQ, bP.reshape(1, h), bQ.reshape(1, h), fP, fQ,
      wuP[:h], wuP[h:], buP.reshape(1, hu), wuQ[:h], wuQ[h:],
      buQ.reshape(1, hu))


def kernel(ufea, vfea, UV_adj, VU_adj, d_gc1_w, d_gc1_b, d_gc2_w, d_gc2_b, d_gc3_w, d_gc3_b, d_gc4_w, d_gc4_b, l_gc1_w, l_gc1_b, l_gc2_w, l_gc2_b, l_gc3m_w, l_gc3m_b, l_gc3s_w, l_gc3s_b, l_gc4m_w, l_gc4m_b, l_gc4s_w, l_gc4s_b, d_uu_w, d_uu_b, d_iu_w, d_iu_b, l_uum_w, l_uum_b, l_uus_w, l_uus_b, l_ium_w, l_ium_b, l_ius_w, l_ius_b):
    y1 = _mm(ufea, d_gc1_w)
    y2 = _mm(vfea, d_gc2_w)
    y3, VU_f8, y4, UV_f8 = _spmm_a2(
        VU_adj, UV_adj, y1, y2, d_gc1_b, d_gc2_b, d_gc3_w, d_gc4_w)
    u, y5, v, y6 = _spmm_b2(
        UV_f8, VU_f8, y3, y4, d_gc3_b, d_gc4_b, ufea, vfea,
        d_uu_w, d_uu_b, d_iu_w, d_iu_b, l_gc1_w, l_gc2_w)
    y7, y8 = _spmm_c2(VU_f8, UV_f8, y5, y6, l_gc1_b, l_gc2_b,
                      l_gc3m_w, l_gc4m_w)
    user, item = _spmm_d2(UV_f8, VU_f8, y7, y8, l_gc3m_b, l_gc4m_b,
                          u, v, l_uum_w, l_ium_w, l_uum_b, l_ium_b)
    return user, item


# speedup vs baseline: 1.0672x; 1.0085x over previous
import jax
import jax.numpy as jnp
from jax.experimental import pallas as pl
from jax.experimental.pallas import tpu as pltpu

_CP = pltpu.CompilerParams(vmem_limit_bytes=64 * 1024 * 1024)

_ALPHA = 0.1
_BF = jnp.bfloat16
_F8 = jnp.float8_e4m3fn


def _pick_blk(n, want):
    return want if n % want == 0 else n


def _dot(a, b):
    return jnp.dot(a, b, preferred_element_type=jnp.float32)


def _lrelu(x):
    return jnp.where(x >= 0.0, x, _ALPHA * x)


def _mm_body(x_ref, w_ref, o_ref):
    o_ref[...] = _dot(x_ref[...], w_ref[...]).astype(o_ref.dtype)


def _mm(x, w):
    n, d = x.shape
    h = w.shape[1]
    blk = _pick_blk(n, 1000)
    return pl.pallas_call(
        _mm_body,
        grid=(n // blk,),
        in_specs=[
            pl.BlockSpec((blk, d), lambda i: (i, 0)),
            pl.BlockSpec((d, h), lambda i: (0, 0)),
        ],
        out_specs=pl.BlockSpec((blk, h), lambda i: (i, 0)),
        compiler_params=_CP,
        out_shape=jax.ShapeDtypeStruct((n, h), _BF),
    )(x, w)


def _spmm_a_body(adj_ref, y_ref, b_ref, wn_ref, yn_ref, adjb_ref):
    a = adj_ref[...].astype(_BF)
    h = _lrelu(_dot(a, y_ref[...]) + b_ref[...])
    yn_ref[...] = _dot(h, wn_ref[...]).astype(_BF)
    adjb_ref[...] = a.astype(_F8)


def _spmm_a(adj, y, b, w_next):
    m, k = adj.shape
    h = y.shape[1]
    hn = w_next.shape[1]
    blk = _pick_blk(m, 400)
    return pl.pallas_call(
        _spmm_a_body,
        grid=(m // blk,),
        in_specs=[
            pl.BlockSpec((blk, k), lambda i: (i, 0)),
            pl.BlockSpec((k, h), lambda i: (0, 0)),
            pl.BlockSpec((1, h), lambda i: (0, 0)),
            pl.BlockSpec((h, hn), lambda i: (0, 0)),
        ],
        out_specs=[
            pl.BlockSpec((blk, hn), lambda i: (i, 0)),
            pl.BlockSpec((blk, k), lambda i: (i, 0)),
        ],
        compiler_params=_CP,
        out_shape=[
            jax.ShapeDtypeStruct((m, hn), _BF),
            jax.ShapeDtypeStruct((m, k), _F8),
        ],
    )(adj, y, b.reshape(1, h), w_next)


def _spmm_b_body(adj_ref, y_ref, b_ref, feat_ref, wu1_ref, wu2_ref, bu_ref,
                 wn_ref, u_ref, yn_ref):
    h = _lrelu(_dot(adj_ref[...], y_ref[...]) + b_ref[...])
    u = _dot(h, wu1_ref[...]) + _dot(feat_ref[...], wu2_ref[...]) + bu_ref[...]
    u = jnp.maximum(u, 0.0)
    u_ref[...] = u
    yn_ref[...] = _dot(u, wn_ref[...]).astype(_BF)


def _spmm_b(adj_bf, y, b, feat, wu, bu, w_next):
    m, k = adj_bf.shape
    h = y.shape[1]
    df = feat.shape[1]
    hu = wu.shape[1]
    hn = w_next.shape[1]
    blk = _pick_blk(m, 1000)
    return pl.pallas_call(
        _spmm_b_body,
        grid=(m // blk,),
        in_specs=[
            pl.BlockSpec((blk, k), lambda i: (i, 0)),
            pl.BlockSpec((k, h), lambda i: (0, 0)),
            pl.BlockSpec((1, h), lambda i: (0, 0)),
            pl.BlockSpec((blk, df), lambda i: (i, 0)),
            pl.BlockSpec((h, hu), lambda i: (0, 0)),
            pl.BlockSpec((df, hu), lambda i: (0, 0)),
            pl.BlockSpec((1, hu), lambda i: (0, 0)),
            pl.BlockSpec((hu, hn), lambda i: (0, 0)),
        ],
        out_specs=[
            pl.BlockSpec((blk, hu), lambda i: (i, 0)),
            pl.BlockSpec((blk, hn), lambda i: (i, 0)),
        ],
        compiler_params=_CP,
        out_shape=[
            jax.ShapeDtypeStruct((m, hu), jnp.float32),
            jax.ShapeDtypeStruct((m, hn), _BF),
        ],
    )(adj_bf, y, b.reshape(1, h), feat, wu[:h], wu[h:], bu.reshape(1, hu),
      w_next)


def _spmm_c_body(adj_ref, y_ref, b_ref, wn_ref, yn_ref):
    h = _lrelu(_dot(adj_ref[...], y_ref[...]) + b_ref[...])
    yn_ref[...] = _dot(h, wn_ref[...]).astype(_BF)


def _spmm_c(adj_bf, y, b, w_next):
    m, k = adj_bf.shape
    h = y.shape[1]
    hn = w_next.shape[1]
    blk = _pick_blk(m, 1000)
    return pl.pallas_call(
        _spmm_c_body,
        grid=(m // blk,),
        in_specs=[
            pl.BlockSpec((blk, k), lambda i: (i, 0)),
            pl.BlockSpec((k, h), lambda i: (0, 0)),
            pl.BlockSpec((1, h), lambda i: (0, 0)),
            pl.BlockSpec((h, hn), lambda i: (0, 0)),
        ],
        out_specs=pl.BlockSpec((blk, hn), lambda i: (i, 0)),
        compiler_params=_CP,
        out_shape=jax.ShapeDtypeStruct((m, hn), _BF),
    )(adj_bf, y, b.reshape(1, h), w_next)


def _spmm_d_body(adj_ref, y_ref, b_ref, feat_ref, wu1_ref, wu2_ref, bu_ref,
                 o_ref):
    h = _lrelu(_dot(adj_ref[...], y_ref[...]) + b_ref[...])
    o_ref[...] = (_dot(h, wu1_ref[...]) + _dot(feat_ref[...], wu2_ref[...])
                  + bu_ref[...])


def _spmm_d(adj_bf, y, b, feat, wu, bu):
    m, k = adj_bf.shape
    h = y.shape[1]
    df = feat.shape[1]
    hu = wu.shape[1]
    blk = _pick_blk(m, 1000)
    return pl.pallas_call(
        _spmm_d_body,
        grid=(m // blk,),
        in_specs=[
            pl.BlockSpec((blk, k), lambda i: (i, 0)),
            pl.BlockSpec((k, h), lambda i: (0, 0)),
            pl.BlockSpec((1, h), lambda i: (0, 0)),
            pl.BlockSpec((blk, df), lambda i: (i, 0)),
            pl.BlockSpec((h, hu), lambda i: (0, 0)),
            pl.BlockSpec((df, hu), lambda i: (0, 0)),
            pl.BlockSpec((1, hu), lambda i: (0, 0)),
        ],
        out_specs=pl.BlockSpec((blk, hu), lambda i: (i, 0)),
        compiler_params=_CP,
        out_shape=jax.ShapeDtypeStruct((m, hu), jnp.float32),
    )(adj_bf, y, b.reshape(1, h), feat, wu[:h], wu[h:], bu.reshape(1, hu))


def kernel(ufea, vfea, UV_adj, VU_adj, d_gc1_w, d_gc1_b, d_gc2_w, d_gc2_b, d_gc3_w, d_gc3_b, d_gc4_w, d_gc4_b, l_gc1_w, l_gc1_b, l_gc2_w, l_gc2_b, l_gc3m_w, l_gc3m_b, l_gc3s_w, l_gc3s_b, l_gc4m_w, l_gc4m_b, l_gc4s_w, l_gc4s_b, d_uu_w, d_uu_b, d_iu_w, d_iu_b, l_uum_w, l_uum_b, l_uus_w, l_uus_b, l_ium_w, l_ium_b, l_ius_w, l_ius_b):
    y1 = _mm(ufea, d_gc1_w)
    y2 = _mm(vfea, d_gc2_w)
    y3, VU_bf = _spmm_a(VU_adj, y1, d_gc1_b, d_gc3_w)
    y4, UV_bf = _spmm_a(UV_adj, y2, d_gc2_b, d_gc4_w)
    u, y5 = _spmm_b(UV_bf, y3, d_gc3_b, ufea, d_uu_w, d_uu_b, l_gc1_w)
    v, y6 = _spmm_b(VU_bf, y4, d_gc4_b, vfea, d_iu_w, d_iu_b, l_gc2_w)
    y7 = _spmm_c(VU_bf, y5, l_gc1_b, l_gc3m_w)
    y8 = _spmm_c(UV_bf, y6, l_gc2_b, l_gc4m_w)
    user = _spmm_d(UV_bf, y7, l_gc3m_b, u, l_uum_w, l_uum_b)
    item = _spmm_d(VU_bf, y8, l_gc4m_b, v, l_ium_w, l_ium_b)
    return user, item
